# baseline (speedup 1.0000x reference)
import functools

import jax
import jax.numpy as jnp
from jax import lax
from jax.experimental import pallas as pl
from jax.experimental.pallas import tpu as pltpu
from jax.experimental.pallas import tpu_sc as plsc

N = 10000
EDGES = 320000
IN_DIM = 128
HEADS = 8
D = 16
HD = HEADS * D

NUM_WORKERS = 32
NPAD = 10240
C = 32
TOTAL_CHUNKS = EDGES // C
IDX_BLK = 16
NBLOCKS = TOTAL_CHUNKS // IDX_BLK
ROWS_PER_TILE = NPAD // 16


def _qkv_body(h_ref, wq_ref, bq_ref, wk_ref, bk_ref, wv_ref, bv_ref,
              q_ref, k_ref, v_ref):
    hb = h_ref[...]
    f32 = jnp.float32
    q_ref[...] = jnp.dot(hb, wq_ref[...], preferred_element_type=f32) + bq_ref[...]
    k_ref[...] = jnp.dot(hb, wk_ref[...], preferred_element_type=f32) + bk_ref[...]
    v_ref[...] = jnp.dot(hb, wv_ref[...], preferred_element_type=f32) + bv_ref[...]


_QKV_BLK = 1000
_qkv_call = pl.pallas_call(
    _qkv_body,
    grid=(N // _QKV_BLK,),
    in_specs=[
        pl.BlockSpec((_QKV_BLK, IN_DIM), lambda i: (i, 0)),
        pl.BlockSpec((IN_DIM, HD), lambda i: (0, 0)),
        pl.BlockSpec((1, HD), lambda i: (0, 0)),
        pl.BlockSpec((IN_DIM, HD), lambda i: (0, 0)),
        pl.BlockSpec((1, HD), lambda i: (0, 0)),
        pl.BlockSpec((IN_DIM, HD), lambda i: (0, 0)),
        pl.BlockSpec((1, HD), lambda i: (0, 0)),
    ],
    out_specs=[
        pl.BlockSpec((_QKV_BLK, HD), lambda i: (i, 0)),
        pl.BlockSpec((_QKV_BLK, HD), lambda i: (i, 0)),
        pl.BlockSpec((_QKV_BLK, HD), lambda i: (i, 0)),
    ],
    out_shape=[
        jax.ShapeDtypeStruct((N, HD), jnp.float32),
        jax.ShapeDtypeStruct((N, HD), jnp.float32),
        jax.ShapeDtypeStruct((N, HD), jnp.float32),
    ],
)


def _ee_body(e_ref, we_ref, be_ref, o_ref):
    o_ref[...] = (jnp.dot(e_ref[...], we_ref[...],
                          preferred_element_type=jnp.float32) + be_ref[...]) * 0.25


_EE_BLK = 2000
_ee_call = pl.pallas_call(
    _ee_body,
    grid=(EDGES // _EE_BLK,),
    in_specs=[
        pl.BlockSpec((_EE_BLK, IN_DIM), lambda i: (i, 0)),
        pl.BlockSpec((IN_DIM, HD), lambda i: (0, 0)),
        pl.BlockSpec((1, HD), lambda i: (0, 0)),
    ],
    out_specs=pl.BlockSpec((_EE_BLK, HD), lambda i: (i, 0)),
    out_shape=jax.ShapeDtypeStruct((EDGES, HD), jnp.float32),
)


def _edge_body(q_hbm, k_hbm, v_hbm, ee_hbm, src_hbm, dst_hbm, outw, outz,
               src_blk, dst_blk, sdst, krows, qrows, erows, vrows, zrows,
               accw, accz, sem_g0, sem_g1, sem_s0, sem_s1):
    c = lax.axis_index("c")
    s = lax.axis_index("s")
    w = c * 16 + s
    lane = lax.iota(jnp.int32, 16)
    perms = [lane ^ sh for sh in (1, 2, 4, 8)]
    zero16 = jnp.zeros((16,), jnp.float32)
    sem_g = (sem_g0, sem_g1)
    sem_s = (sem_s0, sem_s1)

    def _allsum(v):
        for p in perms:
            v = v + v.at[p].get(mode="promise_in_bounds")
        return v

    zero16i = jnp.zeros((16,), jnp.int32)

    def _zq(i, carry):
        for j in range(HD // 16):
            qrows[0, i, pl.ds(j * 16, 16)] = zero16
            vrows[1, i, pl.ds(j * 16, 16)] = zero16
        zrows[0, i, :] = zero16
        zrows[1, i, :] = zero16
        return carry

    lax.fori_loop(0, C, _zq, 0)
    for j in range(C // 16):
        sdst[1, pl.ds(j * 16, 16)] = zero16i

    row0 = s * ROWS_PER_TILE
    for j in range(ROWS_PER_TILE // C):
        pltpu.sync_copy(qrows.at[0], accw.at[pl.ds(row0 + j * C, C)])
        pltpu.sync_copy(zrows.at[0], accz.at[pl.ds(row0 + j * C, C)])
    plsc.subcore_barrier()

    g0 = (w * NBLOCKS) // NUM_WORKERS
    g1 = ((w + 1) * NBLOCKS) // NUM_WORKERS

    def _reload(t):
        bbase = jnp.minimum(t, TOTAL_CHUNKS - IDX_BLK)
        pltpu.sync_copy(src_hbm.at[pl.ds(bbase, IDX_BLK)], src_blk)
        pltpu.sync_copy(dst_hbm.at[pl.ds(bbase, IDX_BLK)], dst_blk)

    def _issue_gathers(r, t, b):
        pltpu.async_copy(k_hbm.at[src_blk.at[r]], krows.at[b], sem_g[b])
        pltpu.async_copy(q_hbm.at[dst_blk.at[r]], qrows.at[b], sem_g[b])
        pltpu.async_copy(ee_hbm.at[pl.ds(t * C, C)], erows.at[b], sem_g[b])
        pltpu.async_copy(v_hbm.at[src_blk.at[r]], vrows.at[b], sem_g[b])

    def _wait_gathers(r, b):
        pltpu.make_async_copy(k_hbm.at[src_blk.at[r]], krows.at[b], sem_g[b]).wait()
        pltpu.make_async_copy(q_hbm.at[dst_blk.at[r]], qrows.at[b], sem_g[b]).wait()
        pltpu.make_async_copy(ee_hbm.at[pl.ds(0, C)], erows.at[b], sem_g[b]).wait()
        pltpu.make_async_copy(v_hbm.at[src_blk.at[r]], vrows.at[b], sem_g[b]).wait()

    def _issue_scatter(r, b):
        for j in range(C // 16):
            sdst[b, pl.ds(j * 16, 16)] = dst_blk[r, pl.ds(j * 16, 16)]
        pltpu.async_copy(vrows.at[b], accw.at[sdst.at[b]], sem_s[b], add=True)
        pltpu.async_copy(zrows.at[b], accz.at[sdst.at[b]], sem_s[b], add=True)

    def _dummy_scatter():
        pltpu.async_copy(vrows.at[1], accw.at[sdst.at[1]], sem_s[1], add=True)
        pltpu.async_copy(zrows.at[1], accz.at[sdst.at[1]], sem_s[1], add=True)

    def _wait_scatter(b):
        pltpu.make_async_copy(vrows.at[b], accw.at[sdst.at[b]], sem_s[b]).wait()
        pltpu.make_async_copy(zrows.at[b], accz.at[sdst.at[b]], sem_s[b]).wait()

    def _compute(b):
        kr, qr, er, vr, zr = (krows.at[b], qrows.at[b], erows.at[b],
                              vrows.at[b], zrows.at[b])

        def _edge(i):
            zrow = zero16
            for hh in range(HEADS):
                sl = pl.ds(hh * 16, 16)
                t3 = kr[i, sl] * qr[i, sl] * er[i, sl]
                sv = jnp.exp(jnp.minimum(jnp.maximum(_allsum(t3), -5.0), 5.0))
                vr[i, sl] = vr[i, sl] * sv
                zrow = jnp.where(lane == hh, sv, zrow)
            zr[i, :] = zrow

        plsc.parallel_loop(0, C, 1, unroll=1)(_edge)

    _dummy_scatter()
    _reload(g0 * IDX_BLK)
    _issue_gathers(0, g0 * IDX_BLK, 0)

    def _block(g, carry):
        tbase = g * IDX_BLK
        for r in range(IDX_BLK):
            b = r & 1
            nb = 1 - b
            _wait_gathers(r, b)
            _compute(b)
            _issue_scatter(r, b)
            _wait_scatter(nb)
            if r == IDX_BLK - 1:
                _reload(tbase + IDX_BLK)
            tn = jnp.minimum(tbase + r + 1, TOTAL_CHUNKS - 1)
            _issue_gathers((r + 1) % IDX_BLK, tn, nb)
        return carry

    lax.fori_loop(g0, g1, _block, 0)

    _wait_gathers(0, 0)
    _wait_scatter(1)

    plsc.subcore_barrier()

    for j in range(ROWS_PER_TILE // C):
        pltpu.sync_copy(accw.at[pl.ds(row0 + j * C, C)], qrows.at[0])
        pltpu.sync_copy(qrows.at[0], outw.at[c, pl.ds(row0 + j * C, C)])
        pltpu.sync_copy(accz.at[pl.ds(row0 + j * C, C)], zrows.at[0])
        pltpu.sync_copy(zrows.at[0], outz.at[c, pl.ds(row0 + j * C, C)])


_edge_call = functools.partial(
    pl.kernel,
    out_type=(
        jax.ShapeDtypeStruct((2, NPAD, HD), jnp.float32),
        jax.ShapeDtypeStruct((2, NPAD, 16), jnp.float32),
    ),
    mesh=plsc.VectorSubcoreMesh(core_axis_name="c", subcore_axis_name="s"),
    compiler_params=pltpu.CompilerParams(use_tc_tiling_on_sc=False),
    scratch_types=[
        pltpu.VMEM((IDX_BLK, C), jnp.int32),
        pltpu.VMEM((IDX_BLK, C), jnp.int32),
        pltpu.VMEM((2, C), jnp.int32),
        pltpu.VMEM((2, C, HD), jnp.float32),
        pltpu.VMEM((2, C, HD), jnp.float32),
        pltpu.VMEM((2, C, HD), jnp.float32),
        pltpu.VMEM((2, C, HD), jnp.float32),
        pltpu.VMEM((2, C, 16), jnp.float32),
        pltpu.VMEM_SHARED((NPAD, HD), jnp.float32),
        pltpu.VMEM_SHARED((NPAD, 16), jnp.float32),
        pltpu.SemaphoreType.DMA,
        pltpu.SemaphoreType.DMA,
        pltpu.SemaphoreType.DMA,
        pltpu.SemaphoreType.DMA,
    ],
)(_edge_body)


def _comb_body(aw_ref, az_ref, o_ref):
    wsum = aw_ref[0] + aw_ref[1]
    zsum = az_ref[0] + az_ref[1] + 1e-6
    for hh in range(HEADS):
        o_ref[:, hh * 16:(hh + 1) * 16] = (
            wsum[:, hh * 16:(hh + 1) * 16] / zsum[:, hh:hh + 1])


_COMB_BLK = 1000
_comb_call = pl.pallas_call(
    _comb_body,
    grid=(N // _COMB_BLK,),
    in_specs=[
        pl.BlockSpec((2, _COMB_BLK, HD), lambda i: (0, i, 0)),
        pl.BlockSpec((2, _COMB_BLK, 16), lambda i: (0, i, 0)),
    ],
    out_specs=pl.BlockSpec((_COMB_BLK, HD), lambda i: (i, 0)),
    out_shape=jax.ShapeDtypeStruct((N, HD), jnp.float32),
)


def kernel(h, e, edge_index, Wq, bq, Wk, bk, We, be, Wv, bv):
    q, k, v = _qkv_call(h, Wq, bq.reshape(1, HD), Wk, bk.reshape(1, HD),
                        Wv, bv.reshape(1, HD))
    ee = _ee_call(e, We, be.reshape(1, HD))
    src2 = edge_index[0].reshape(TOTAL_CHUNKS, C)
    dst2 = edge_index[1].reshape(TOTAL_CHUNKS, C)
    accw, accz = _edge_call(q, k, v, ee, src2, dst2)
    out = _comb_call(accw, accz)
    return out.reshape(N, HEADS, D)


# speedup vs baseline: 1.3501x; 1.3501x over previous
import functools

import jax
import jax.numpy as jnp
from jax import lax
from jax.experimental import pallas as pl
from jax.experimental.pallas import tpu as pltpu
from jax.experimental.pallas import tpu_sc as plsc

N = 10000
EDGES = 320000
IN_DIM = 128
HEADS = 8
D = 16
HD = HEADS * D

NUM_WORKERS = 32
NPAD = 10240
C = 32
TOTAL_CHUNKS = EDGES // C
IDX_BLK = 16
NBLOCKS = TOTAL_CHUNKS // IDX_BLK
ROWS_PER_TILE = NPAD // 16


def _qkv_body(h_ref, wq_ref, bq_ref, wk_ref, bk_ref, wv_ref, bv_ref,
              q_ref, k_ref, v_ref):
    hb = h_ref[...]
    f32 = jnp.float32
    q_ref[...] = jnp.dot(hb, wq_ref[...], preferred_element_type=f32) + bq_ref[...]
    k_ref[...] = jnp.dot(hb, wk_ref[...], preferred_element_type=f32) + bk_ref[...]
    v_ref[...] = jnp.dot(hb, wv_ref[...], preferred_element_type=f32) + bv_ref[...]


_QKV_BLK = 1000
_qkv_call = pl.pallas_call(
    _qkv_body,
    grid=(N // _QKV_BLK,),
    in_specs=[
        pl.BlockSpec((_QKV_BLK, IN_DIM), lambda i: (i, 0)),
        pl.BlockSpec((IN_DIM, HD), lambda i: (0, 0)),
        pl.BlockSpec((1, HD), lambda i: (0, 0)),
        pl.BlockSpec((IN_DIM, HD), lambda i: (0, 0)),
        pl.BlockSpec((1, HD), lambda i: (0, 0)),
        pl.BlockSpec((IN_DIM, HD), lambda i: (0, 0)),
        pl.BlockSpec((1, HD), lambda i: (0, 0)),
    ],
    out_specs=[
        pl.BlockSpec((_QKV_BLK, HD), lambda i: (i, 0)),
        pl.BlockSpec((_QKV_BLK, HD), lambda i: (i, 0)),
        pl.BlockSpec((_QKV_BLK, HD), lambda i: (i, 0)),
    ],
    out_shape=[
        jax.ShapeDtypeStruct((N, HD), jnp.float32),
        jax.ShapeDtypeStruct((N, HD), jnp.float32),
        jax.ShapeDtypeStruct((N, HD), jnp.float32),
    ],
)


def _ee_body(e_ref, we_ref, be_ref, o_ref):
    o_ref[...] = (jnp.dot(e_ref[...], we_ref[...],
                          preferred_element_type=jnp.float32) + be_ref[...]) * 0.25


_EE_BLK = 2000
_ee_call = pl.pallas_call(
    _ee_body,
    grid=(EDGES // _EE_BLK,),
    in_specs=[
        pl.BlockSpec((_EE_BLK, IN_DIM), lambda i: (i, 0)),
        pl.BlockSpec((IN_DIM, HD), lambda i: (0, 0)),
        pl.BlockSpec((1, HD), lambda i: (0, 0)),
    ],
    out_specs=pl.BlockSpec((_EE_BLK, HD), lambda i: (i, 0)),
    out_shape=jax.ShapeDtypeStruct((EDGES, HD), jnp.float32),
)


def _edge_body(q_hbm, k_hbm, v_hbm, ee_hbm, src_hbm, dst_hbm, outw, outz,
               src_blk, dst_blk, sdst, krows, qrows, erows, vrows, zrows,
               accw, accz, sem_g0, sem_g1, sem_s0, sem_s1):
    c = lax.axis_index("c")
    s = lax.axis_index("s")
    w = c * 16 + s
    lane = lax.iota(jnp.int32, 16)
    perms = [lane ^ sh for sh in (1, 2, 4, 8)]
    zero16 = jnp.zeros((16,), jnp.float32)
    sem_g = (sem_g0, sem_g1)
    sem_s = (sem_s0, sem_s1)

    def _allsum(v):
        for p in perms:
            v = v + v.at[p].get(mode="promise_in_bounds")
        return v

    zero16i = jnp.zeros((16,), jnp.int32)

    def _zq(i, carry):
        for j in range(HD // 16):
            qrows[0, i, pl.ds(j * 16, 16)] = zero16
            vrows[1, i, pl.ds(j * 16, 16)] = zero16
        zrows[0, i, :] = zero16
        zrows[1, i, :] = zero16
        return carry

    lax.fori_loop(0, C, _zq, 0)
    for j in range(C // 16):
        sdst[1, pl.ds(j * 16, 16)] = zero16i

    row0 = s * ROWS_PER_TILE
    for j in range(ROWS_PER_TILE // C):
        pltpu.sync_copy(qrows.at[0], accw.at[pl.ds(row0 + j * C, C)])
        pltpu.sync_copy(zrows.at[0], accz.at[pl.ds(row0 + j * C, C)])
    plsc.subcore_barrier()

    g0 = (w * NBLOCKS) // NUM_WORKERS
    g1 = ((w + 1) * NBLOCKS) // NUM_WORKERS

    def _reload(t):
        bbase = jnp.minimum(t, TOTAL_CHUNKS - IDX_BLK)
        pltpu.sync_copy(src_hbm.at[pl.ds(bbase, IDX_BLK)], src_blk)
        pltpu.sync_copy(dst_hbm.at[pl.ds(bbase, IDX_BLK)], dst_blk)

    def _issue_gathers(r, t, b):
        pltpu.async_copy(k_hbm.at[src_blk.at[r]], krows.at[b], sem_g[b])
        pltpu.async_copy(q_hbm.at[dst_blk.at[r]], qrows.at[b], sem_g[b])
        pltpu.async_copy(ee_hbm.at[pl.ds(t * C, C)], erows.at[b], sem_g[b])
        pltpu.async_copy(v_hbm.at[src_blk.at[r]], vrows.at[b], sem_g[b])

    def _wait_gathers(r, b):
        pltpu.make_async_copy(k_hbm.at[src_blk.at[r]], krows.at[b], sem_g[b]).wait()
        pltpu.make_async_copy(q_hbm.at[dst_blk.at[r]], qrows.at[b], sem_g[b]).wait()
        pltpu.make_async_copy(ee_hbm.at[pl.ds(0, C)], erows.at[b], sem_g[b]).wait()
        pltpu.make_async_copy(v_hbm.at[src_blk.at[r]], vrows.at[b], sem_g[b]).wait()

    def _snap_dst(r, b):
        for j in range(C // 16):
            sdst[b, pl.ds(j * 16, 16)] = dst_blk[r, pl.ds(j * 16, 16)]

    def _issue_scatter(b):
        pltpu.async_copy(vrows.at[b], accw.at[sdst.at[b]], sem_s[b], add=True)
        pltpu.async_copy(zrows.at[b], accz.at[sdst.at[b]], sem_s[b], add=True)

    def _dummy_scatter():
        pltpu.async_copy(vrows.at[1], accw.at[sdst.at[1]], sem_s[1], add=True)
        pltpu.async_copy(zrows.at[1], accz.at[sdst.at[1]], sem_s[1], add=True)

    def _wait_scatter(b):
        pltpu.make_async_copy(vrows.at[b], accw.at[sdst.at[b]], sem_s[b]).wait()
        pltpu.make_async_copy(zrows.at[b], accz.at[sdst.at[b]], sem_s[b]).wait()

    def _compute(b):
        kr, qr, er, vr, zr = (krows.at[b], qrows.at[b], erows.at[b],
                              vrows.at[b], zrows.at[b])

        def _edge(i):
            zrow = zero16
            for hh in range(HEADS):
                sl = pl.ds(hh * 16, 16)
                t3 = kr[i, sl] * qr[i, sl] * er[i, sl]
                sv = jnp.exp(jnp.minimum(jnp.maximum(_allsum(t3), -5.0), 5.0))
                vr[i, sl] = vr[i, sl] * sv
                zrow = jnp.where(lane == hh, sv, zrow)
            zr[i, :] = zrow

        plsc.parallel_loop(0, C, 1, unroll=1)(_edge)

    _dummy_scatter()
    _reload(g0 * IDX_BLK)
    _issue_gathers(0, g0 * IDX_BLK, 0)

    def _block(g, carry):
        tbase = g * IDX_BLK
        for r in range(IDX_BLK):
            b = r & 1
            nb = 1 - b
            _wait_gathers(r, b)
            _snap_dst(r, b)
            _wait_scatter(nb)
            if r == IDX_BLK - 1:
                _reload(tbase + IDX_BLK)
            tn = jnp.minimum(tbase + r + 1, TOTAL_CHUNKS - 1)
            _issue_gathers((r + 1) % IDX_BLK, tn, nb)
            _compute(b)
            _issue_scatter(b)
        return carry

    lax.fori_loop(g0, g1, _block, 0)

    _wait_gathers(0, 0)
    _wait_scatter(1)

    plsc.subcore_barrier()

    for j in range(ROWS_PER_TILE // C):
        pltpu.sync_copy(accw.at[pl.ds(row0 + j * C, C)], qrows.at[0])
        pltpu.sync_copy(qrows.at[0], outw.at[c, pl.ds(row0 + j * C, C)])
        pltpu.sync_copy(accz.at[pl.ds(row0 + j * C, C)], zrows.at[0])
        pltpu.sync_copy(zrows.at[0], outz.at[c, pl.ds(row0 + j * C, C)])


_edge_call = functools.partial(
    pl.kernel,
    out_type=(
        jax.ShapeDtypeStruct((2, NPAD, HD), jnp.float32),
        jax.ShapeDtypeStruct((2, NPAD, 16), jnp.float32),
    ),
    mesh=plsc.VectorSubcoreMesh(core_axis_name="c", subcore_axis_name="s"),
    compiler_params=pltpu.CompilerParams(use_tc_tiling_on_sc=False),
    scratch_types=[
        pltpu.VMEM((IDX_BLK, C), jnp.int32),
        pltpu.VMEM((IDX_BLK, C), jnp.int32),
        pltpu.VMEM((2, C), jnp.int32),
        pltpu.VMEM((2, C, HD), jnp.float32),
        pltpu.VMEM((2, C, HD), jnp.float32),
        pltpu.VMEM((2, C, HD), jnp.float32),
        pltpu.VMEM((2, C, HD), jnp.float32),
        pltpu.VMEM((2, C, 16), jnp.float32),
        pltpu.VMEM_SHARED((NPAD, HD), jnp.float32),
        pltpu.VMEM_SHARED((NPAD, 16), jnp.float32),
        pltpu.SemaphoreType.DMA,
        pltpu.SemaphoreType.DMA,
        pltpu.SemaphoreType.DMA,
        pltpu.SemaphoreType.DMA,
    ],
)(_edge_body)


def _comb_body(aw_ref, az_ref, o_ref):
    wsum = aw_ref[0] + aw_ref[1]
    zsum = az_ref[0] + az_ref[1] + 1e-6
    for hh in range(HEADS):
        o_ref[:, hh * 16:(hh + 1) * 16] = (
            wsum[:, hh * 16:(hh + 1) * 16] / zsum[:, hh:hh + 1])


_COMB_BLK = 1000
_comb_call = pl.pallas_call(
    _comb_body,
    grid=(N // _COMB_BLK,),
    in_specs=[
        pl.BlockSpec((2, _COMB_BLK, HD), lambda i: (0, i, 0)),
        pl.BlockSpec((2, _COMB_BLK, 16), lambda i: (0, i, 0)),
    ],
    out_specs=pl.BlockSpec((_COMB_BLK, HD), lambda i: (i, 0)),
    out_shape=jax.ShapeDtypeStruct((N, HD), jnp.float32),
)


def kernel(h, e, edge_index, Wq, bq, Wk, bk, We, be, Wv, bv):
    q, k, v = _qkv_call(h, Wq, bq.reshape(1, HD), Wk, bk.reshape(1, HD),
                        Wv, bv.reshape(1, HD))
    ee = _ee_call(e, We, be.reshape(1, HD))
    src2 = edge_index[0].reshape(TOTAL_CHUNKS, C)
    dst2 = edge_index[1].reshape(TOTAL_CHUNKS, C)
    accw, accz = _edge_call(q, k, v, ee, src2, dst2)
    out = _comb_call(accw, accz)
    return out.reshape(N, HEADS, D)


# speedup vs baseline: 1.4573x; 1.0793x over previous
import functools

import jax
import jax.numpy as jnp
from jax import lax
from jax.experimental import pallas as pl
from jax.experimental.pallas import tpu as pltpu
from jax.experimental.pallas import tpu_sc as plsc

N = 10000
EDGES = 320000
IN_DIM = 128
HEADS = 8
D = 16
HD = HEADS * D

NUM_WORKERS = 32
NPAD = 10240
C = 32
TOTAL_CHUNKS = EDGES // C
IDX_BLK = 16
NBLOCKS = TOTAL_CHUNKS // IDX_BLK
ROWS_PER_TILE = NPAD // 16


def _qkv_body(h_ref, wq_ref, bq_ref, wk_ref, bk_ref, wv_ref, bv_ref,
              q_ref, k_ref, v_ref):
    hb = h_ref[...]
    f32 = jnp.float32
    q_ref[...] = jnp.dot(hb, wq_ref[...], preferred_element_type=f32) + bq_ref[...]
    k_ref[...] = jnp.dot(hb, wk_ref[...], preferred_element_type=f32) + bk_ref[...]
    v_ref[...] = jnp.dot(hb, wv_ref[...], preferred_element_type=f32) + bv_ref[...]


_QKV_BLK = 2000
_qkv_call = pl.pallas_call(
    _qkv_body,
    grid=(N // _QKV_BLK,),
    in_specs=[
        pl.BlockSpec((_QKV_BLK, IN_DIM), lambda i: (i, 0)),
        pl.BlockSpec((IN_DIM, HD), lambda i: (0, 0)),
        pl.BlockSpec((1, HD), lambda i: (0, 0)),
        pl.BlockSpec((IN_DIM, HD), lambda i: (0, 0)),
        pl.BlockSpec((1, HD), lambda i: (0, 0)),
        pl.BlockSpec((IN_DIM, HD), lambda i: (0, 0)),
        pl.BlockSpec((1, HD), lambda i: (0, 0)),
    ],
    out_specs=[
        pl.BlockSpec((_QKV_BLK, HD), lambda i: (i, 0)),
        pl.BlockSpec((_QKV_BLK, HD), lambda i: (i, 0)),
        pl.BlockSpec((_QKV_BLK, HD), lambda i: (i, 0)),
    ],
    out_shape=[
        jax.ShapeDtypeStruct((N, HD), jnp.float32),
        jax.ShapeDtypeStruct((N, HD), jnp.float32),
        jax.ShapeDtypeStruct((N, HD), jnp.float32),
    ],
)


def _ee_body(e_ref, we_ref, be_ref, o_ref):
    o_ref[...] = (jnp.dot(e_ref[...], we_ref[...],
                          preferred_element_type=jnp.float32) + be_ref[...]) * 0.25


_EE_BLK = 4000
_ee_call = pl.pallas_call(
    _ee_body,
    grid=(EDGES // _EE_BLK,),
    in_specs=[
        pl.BlockSpec((_EE_BLK, IN_DIM), lambda i: (i, 0)),
        pl.BlockSpec((IN_DIM, HD), lambda i: (0, 0)),
        pl.BlockSpec((1, HD), lambda i: (0, 0)),
    ],
    out_specs=pl.BlockSpec((_EE_BLK, HD), lambda i: (i, 0)),
    out_shape=jax.ShapeDtypeStruct((EDGES, HD), jnp.float32),
)


def _edge_body(q_hbm, k_hbm, v_hbm, ee_hbm, src_hbm, dst_hbm, outw, outz,
               src_blk, dst_blk, sdst, krows, qrows, erows, vrows, zrows,
               accw, accz, sem_g0, sem_g1, sem_s0, sem_s1):
    c = lax.axis_index("c")
    s = lax.axis_index("s")
    w = c * 16 + s
    lane = lax.iota(jnp.int32, 16)
    perms = [lane ^ sh for sh in (1, 2, 4, 8)]
    zero16 = jnp.zeros((16,), jnp.float32)
    sem_g = (sem_g0, sem_g1)
    sem_s = (sem_s0, sem_s1)

    def _allsum(v):
        for p in perms:
            v = v + v.at[p].get(mode="promise_in_bounds")
        return v

    zero16i = jnp.zeros((16,), jnp.int32)

    def _zq(i, carry):
        for j in range(HD // 16):
            qrows[0, i, pl.ds(j * 16, 16)] = zero16
            vrows[1, i, pl.ds(j * 16, 16)] = zero16
        zrows[0, i, :] = zero16
        zrows[1, i, :] = zero16
        return carry

    lax.fori_loop(0, C, _zq, 0)
    for j in range(C // 16):
        sdst[1, pl.ds(j * 16, 16)] = zero16i

    row0 = s * ROWS_PER_TILE
    for j in range(ROWS_PER_TILE // C):
        pltpu.sync_copy(qrows.at[0], accw.at[pl.ds(row0 + j * C, C)])
        pltpu.sync_copy(zrows.at[0], accz.at[pl.ds(row0 + j * C, C)])
    plsc.subcore_barrier()

    g0 = (w * NBLOCKS) // NUM_WORKERS
    g1 = ((w + 1) * NBLOCKS) // NUM_WORKERS

    def _reload(t):
        bbase = jnp.minimum(t, TOTAL_CHUNKS - IDX_BLK)
        pltpu.sync_copy(src_hbm.at[pl.ds(bbase, IDX_BLK)], src_blk)
        pltpu.sync_copy(dst_hbm.at[pl.ds(bbase, IDX_BLK)], dst_blk)

    def _issue_gathers(r, t, b):
        pltpu.async_copy(k_hbm.at[src_blk.at[r]], krows.at[b], sem_g[b])
        pltpu.async_copy(q_hbm.at[dst_blk.at[r]], qrows.at[b], sem_g[b])
        pltpu.async_copy(ee_hbm.at[pl.ds(t * C, C)], erows.at[b], sem_g[b])
        pltpu.async_copy(v_hbm.at[src_blk.at[r]], vrows.at[b], sem_g[b])

    def _wait_gathers(r, b):
        pltpu.make_async_copy(k_hbm.at[src_blk.at[r]], krows.at[b], sem_g[b]).wait()
        pltpu.make_async_copy(q_hbm.at[dst_blk.at[r]], qrows.at[b], sem_g[b]).wait()
        pltpu.make_async_copy(ee_hbm.at[pl.ds(0, C)], erows.at[b], sem_g[b]).wait()
        pltpu.make_async_copy(v_hbm.at[src_blk.at[r]], vrows.at[b], sem_g[b]).wait()

    def _snap_dst(r, b):
        for j in range(C // 16):
            sdst[b, pl.ds(j * 16, 16)] = dst_blk[r, pl.ds(j * 16, 16)]

    def _issue_scatter(b):
        pltpu.async_copy(vrows.at[b], accw.at[sdst.at[b]], sem_s[b], add=True)
        pltpu.async_copy(zrows.at[b], accz.at[sdst.at[b]], sem_s[b], add=True)

    def _dummy_scatter():
        pltpu.async_copy(vrows.at[1], accw.at[sdst.at[1]], sem_s[1], add=True)
        pltpu.async_copy(zrows.at[1], accz.at[sdst.at[1]], sem_s[1], add=True)

    def _wait_scatter(b):
        pltpu.make_async_copy(vrows.at[b], accw.at[sdst.at[b]], sem_s[b]).wait()
        pltpu.make_async_copy(zrows.at[b], accz.at[sdst.at[b]], sem_s[b]).wait()

    def _compute(b):
        kr, qr, er, vr, zr = (krows.at[b], qrows.at[b], erows.at[b],
                              vrows.at[b], zrows.at[b])

        def _edge(i):
            zrow = zero16
            for hh in range(HEADS):
                sl = pl.ds(hh * 16, 16)
                t3 = kr[i, sl] * qr[i, sl] * er[i, sl]
                sv = jnp.exp(jnp.minimum(jnp.maximum(_allsum(t3), -5.0), 5.0))
                vr[i, sl] = vr[i, sl] * sv
                zrow = jnp.where(lane == hh, sv, zrow)
            zr[i, :] = zrow

        plsc.parallel_loop(0, C, 1, unroll=1)(_edge)

    _dummy_scatter()
    _reload(g0 * IDX_BLK)
    _issue_gathers(0, g0 * IDX_BLK, 0)

    def _block(g, carry):
        tbase = g * IDX_BLK
        for r in range(IDX_BLK):
            b = r & 1
            nb = 1 - b
            _wait_gathers(r, b)
            _snap_dst(r, b)
            _wait_scatter(nb)
            if r == IDX_BLK - 1:
                _reload(tbase + IDX_BLK)
            tn = jnp.minimum(tbase + r + 1, TOTAL_CHUNKS - 1)
            _issue_gathers((r + 1) % IDX_BLK, tn, nb)
            _compute(b)
            _issue_scatter(b)
        return carry

    lax.fori_loop(g0, g1, _block, 0)

    _wait_gathers(0, 0)
    _wait_scatter(1)

    plsc.subcore_barrier()

    for j in range(ROWS_PER_TILE // C):
        pltpu.sync_copy(accw.at[pl.ds(row0 + j * C, C)], qrows.at[0])
        pltpu.sync_copy(qrows.at[0], outw.at[c, pl.ds(row0 + j * C, C)])
        pltpu.sync_copy(accz.at[pl.ds(row0 + j * C, C)], zrows.at[0])
        pltpu.sync_copy(zrows.at[0], outz.at[c, pl.ds(row0 + j * C, C)])


_edge_call = functools.partial(
    pl.kernel,
    out_type=(
        jax.ShapeDtypeStruct((2, NPAD, HD), jnp.float32),
        jax.ShapeDtypeStruct((2, NPAD, 16), jnp.float32),
    ),
    mesh=plsc.VectorSubcoreMesh(core_axis_name="c", subcore_axis_name="s"),
    compiler_params=pltpu.CompilerParams(use_tc_tiling_on_sc=False),
    scratch_types=[
        pltpu.VMEM((IDX_BLK, C), jnp.int32),
        pltpu.VMEM((IDX_BLK, C), jnp.int32),
        pltpu.VMEM((2, C), jnp.int32),
        pltpu.VMEM((2, C, HD), jnp.float32),
        pltpu.VMEM((2, C, HD), jnp.float32),
        pltpu.VMEM((2, C, HD), jnp.float32),
        pltpu.VMEM((2, C, HD), jnp.float32),
        pltpu.VMEM((2, C, 16), jnp.float32),
        pltpu.VMEM_SHARED((NPAD, HD), jnp.float32),
        pltpu.VMEM_SHARED((NPAD, 16), jnp.float32),
        pltpu.SemaphoreType.DMA,
        pltpu.SemaphoreType.DMA,
        pltpu.SemaphoreType.DMA,
        pltpu.SemaphoreType.DMA,
    ],
)(_edge_body)


def _comb_body(aw_ref, az_ref, o_ref):
    wsum = aw_ref[0] + aw_ref[1]
    zsum = az_ref[0] + az_ref[1] + 1e-6
    for hh in range(HEADS):
        o_ref[:, hh * 16:(hh + 1) * 16] = (
            wsum[:, hh * 16:(hh + 1) * 16] / zsum[:, hh:hh + 1])


_COMB_BLK = 1000
_comb_call = pl.pallas_call(
    _comb_body,
    grid=(N // _COMB_BLK,),
    in_specs=[
        pl.BlockSpec((2, _COMB_BLK, HD), lambda i: (0, i, 0)),
        pl.BlockSpec((2, _COMB_BLK, 16), lambda i: (0, i, 0)),
    ],
    out_specs=pl.BlockSpec((_COMB_BLK, HD), lambda i: (i, 0)),
    out_shape=jax.ShapeDtypeStruct((N, HD), jnp.float32),
)


def kernel(h, e, edge_index, Wq, bq, Wk, bk, We, be, Wv, bv):
    q, k, v = _qkv_call(h, Wq, bq.reshape(1, HD), Wk, bk.reshape(1, HD),
                        Wv, bv.reshape(1, HD))
    ee = _ee_call(e, We, be.reshape(1, HD))
    src2 = edge_index[0].reshape(TOTAL_CHUNKS, C)
    dst2 = edge_index[1].reshape(TOTAL_CHUNKS, C)
    accw, accz = _edge_call(q, k, v, ee, src2, dst2)
    out = _comb_call(accw, accz)
    return out.reshape(N, HEADS, D)


# speedup vs baseline: 1.4985x; 1.0283x over previous
import functools

import jax
import jax.numpy as jnp
from jax import lax
from jax.experimental import pallas as pl
from jax.experimental.pallas import tpu as pltpu
from jax.experimental.pallas import tpu_sc as plsc

N = 10000
EDGES = 320000
IN_DIM = 128
HEADS = 8
D = 16
HD = HEADS * D

NUM_WORKERS = 32
NPAD = 10240
C = 32
TOTAL_CHUNKS = EDGES // C
IDX_BLK = 16
NBLOCKS = TOTAL_CHUNKS // IDX_BLK
ROWS_PER_TILE = NPAD // 16


def _qkv_body(h_ref, wq_ref, bq_ref, wk_ref, bk_ref, wv_ref, bv_ref,
              q_ref, k_ref, v_ref):
    hb = h_ref[...]
    f32 = jnp.float32
    q_ref[...] = jnp.dot(hb, wq_ref[...], preferred_element_type=f32) + bq_ref[...]
    k_ref[...] = jnp.dot(hb, wk_ref[...], preferred_element_type=f32) + bk_ref[...]
    v_ref[...] = jnp.dot(hb, wv_ref[...], preferred_element_type=f32) + bv_ref[...]


_QKV_BLK = 2000
_qkv_call = pl.pallas_call(
    _qkv_body,
    grid=(N // _QKV_BLK,),
    in_specs=[
        pl.BlockSpec((_QKV_BLK, IN_DIM), lambda i: (i, 0)),
        pl.BlockSpec((IN_DIM, HD), lambda i: (0, 0)),
        pl.BlockSpec((1, HD), lambda i: (0, 0)),
        pl.BlockSpec((IN_DIM, HD), lambda i: (0, 0)),
        pl.BlockSpec((1, HD), lambda i: (0, 0)),
        pl.BlockSpec((IN_DIM, HD), lambda i: (0, 0)),
        pl.BlockSpec((1, HD), lambda i: (0, 0)),
    ],
    out_specs=[
        pl.BlockSpec((_QKV_BLK, HD), lambda i: (i, 0)),
        pl.BlockSpec((_QKV_BLK, HD), lambda i: (i, 0)),
        pl.BlockSpec((_QKV_BLK, HD), lambda i: (i, 0)),
    ],
    out_shape=[
        jax.ShapeDtypeStruct((N, HD), jnp.float32),
        jax.ShapeDtypeStruct((N, HD), jnp.float32),
        jax.ShapeDtypeStruct((N, HD), jnp.float32),
    ],
)


def _ee_body(e_ref, we_ref, be_ref, o_ref):
    o_ref[...] = (jnp.dot(e_ref[...], we_ref[...],
                          preferred_element_type=jnp.float32) + be_ref[...]) * 0.25


_EE_BLK = 8000
_ee_call = pl.pallas_call(
    _ee_body,
    grid=(EDGES // _EE_BLK,),
    in_specs=[
        pl.BlockSpec((_EE_BLK, IN_DIM), lambda i: (i, 0)),
        pl.BlockSpec((IN_DIM, HD), lambda i: (0, 0)),
        pl.BlockSpec((1, HD), lambda i: (0, 0)),
    ],
    out_specs=pl.BlockSpec((_EE_BLK, HD), lambda i: (i, 0)),
    out_shape=jax.ShapeDtypeStruct((EDGES, HD), jnp.float32),
)


def _edge_body(q_hbm, k_hbm, v_hbm, ee_hbm, src_hbm, dst_hbm, outw, outz,
               src_blk, dst_blk, sdst, krows, qrows, erows, vrows, zrows,
               accw, accz, sem_g0, sem_g1, sem_s0, sem_s1):
    c = lax.axis_index("c")
    s = lax.axis_index("s")
    w = c * 16 + s
    lane = lax.iota(jnp.int32, 16)
    perms = [lane ^ sh for sh in (1, 2, 4, 8)]
    zero16 = jnp.zeros((16,), jnp.float32)
    sem_g = (sem_g0, sem_g1)
    sem_s = (sem_s0, sem_s1)

    def _allsum(v):
        for p in perms:
            v = v + v.at[p].get(mode="promise_in_bounds")
        return v

    zero16i = jnp.zeros((16,), jnp.int32)

    def _zq(i, carry):
        for j in range(HD // 16):
            qrows[0, i, pl.ds(j * 16, 16)] = zero16
            vrows[1, i, pl.ds(j * 16, 16)] = zero16
        zrows[0, i, :] = zero16
        zrows[1, i, :] = zero16
        return carry

    lax.fori_loop(0, C, _zq, 0)
    for j in range(C // 16):
        sdst[1, pl.ds(j * 16, 16)] = zero16i

    row0 = s * ROWS_PER_TILE
    for j in range(ROWS_PER_TILE // C):
        pltpu.sync_copy(qrows.at[0], accw.at[pl.ds(row0 + j * C, C)])
        pltpu.sync_copy(zrows.at[0], accz.at[pl.ds(row0 + j * C, C)])
    plsc.subcore_barrier()

    g0 = (w * NBLOCKS) // NUM_WORKERS
    g1 = ((w + 1) * NBLOCKS) // NUM_WORKERS

    def _reload(t):
        bbase = jnp.minimum(t, TOTAL_CHUNKS - IDX_BLK)
        pltpu.sync_copy(src_hbm.at[pl.ds(bbase, IDX_BLK)], src_blk)
        pltpu.sync_copy(dst_hbm.at[pl.ds(bbase, IDX_BLK)], dst_blk)

    def _issue_gathers(r, t, b):
        pltpu.async_copy(k_hbm.at[src_blk.at[r]], krows.at[b], sem_g[b])
        pltpu.async_copy(q_hbm.at[dst_blk.at[r]], qrows.at[b], sem_g[b])
        pltpu.async_copy(ee_hbm.at[pl.ds(t * C, C)], erows.at[b], sem_g[b])
        pltpu.async_copy(v_hbm.at[src_blk.at[r]], vrows.at[b], sem_g[b])

    def _wait_gathers(r, b):
        pltpu.make_async_copy(k_hbm.at[src_blk.at[r]], krows.at[b], sem_g[b]).wait()
        pltpu.make_async_copy(q_hbm.at[dst_blk.at[r]], qrows.at[b], sem_g[b]).wait()
        pltpu.make_async_copy(ee_hbm.at[pl.ds(0, C)], erows.at[b], sem_g[b]).wait()
        pltpu.make_async_copy(v_hbm.at[src_blk.at[r]], vrows.at[b], sem_g[b]).wait()

    def _snap_dst(r, b):
        for j in range(C // 16):
            sdst[b, pl.ds(j * 16, 16)] = dst_blk[r, pl.ds(j * 16, 16)]

    def _issue_scatter(b):
        pltpu.async_copy(vrows.at[b], accw.at[sdst.at[b]], sem_s[b], add=True)
        pltpu.async_copy(zrows.at[b], accz.at[sdst.at[b]], sem_s[b], add=True)

    def _dummy_scatter():
        pltpu.async_copy(vrows.at[1], accw.at[sdst.at[1]], sem_s[1], add=True)
        pltpu.async_copy(zrows.at[1], accz.at[sdst.at[1]], sem_s[1], add=True)

    def _wait_scatter(b):
        pltpu.make_async_copy(vrows.at[b], accw.at[sdst.at[b]], sem_s[b]).wait()
        pltpu.make_async_copy(zrows.at[b], accz.at[sdst.at[b]], sem_s[b]).wait()

    def _compute(b):
        kr, qr, er, vr, zr = (krows.at[b], qrows.at[b], erows.at[b],
                              vrows.at[b], zrows.at[b])

        def _edge(i):
            zrow = zero16
            for hh in range(HEADS):
                sl = pl.ds(hh * 16, 16)
                t3 = kr[i, sl] * qr[i, sl] * er[i, sl]
                sv = jnp.exp(jnp.minimum(jnp.maximum(_allsum(t3), -5.0), 5.0))
                vr[i, sl] = vr[i, sl] * sv
                zrow = jnp.where(lane == hh, sv, zrow)
            zr[i, :] = zrow

        plsc.parallel_loop(0, C, 1, unroll=1)(_edge)

    _dummy_scatter()
    _reload(g0 * IDX_BLK)
    _issue_gathers(0, g0 * IDX_BLK, 0)

    def _block(g, carry):
        tbase = g * IDX_BLK
        for r in range(IDX_BLK):
            b = r & 1
            nb = 1 - b
            _wait_gathers(r, b)
            _snap_dst(r, b)
            _wait_scatter(nb)
            if r == IDX_BLK - 1:
                _reload(tbase + IDX_BLK)
            tn = jnp.minimum(tbase + r + 1, TOTAL_CHUNKS - 1)
            _issue_gathers((r + 1) % IDX_BLK, tn, nb)
            _compute(b)
            _issue_scatter(b)
        return carry

    lax.fori_loop(g0, g1, _block, 0)

    _wait_gathers(0, 0)
    _wait_scatter(1)

    plsc.subcore_barrier()

    for j in range(ROWS_PER_TILE // C):
        pltpu.sync_copy(accw.at[pl.ds(row0 + j * C, C)], qrows.at[0])
        pltpu.sync_copy(qrows.at[0], outw.at[c, pl.ds(row0 + j * C, C)])
        pltpu.sync_copy(accz.at[pl.ds(row0 + j * C, C)], zrows.at[0])
        pltpu.sync_copy(zrows.at[0], outz.at[c, pl.ds(row0 + j * C, C)])


_edge_call = functools.partial(
    pl.kernel,
    out_type=(
        jax.ShapeDtypeStruct((2, NPAD, HD), jnp.float32),
        jax.ShapeDtypeStruct((2, NPAD, 16), jnp.float32),
    ),
    mesh=plsc.VectorSubcoreMesh(core_axis_name="c", subcore_axis_name="s"),
    compiler_params=pltpu.CompilerParams(use_tc_tiling_on_sc=False),
    scratch_types=[
        pltpu.VMEM((IDX_BLK, C), jnp.int32),
        pltpu.VMEM((IDX_BLK, C), jnp.int32),
        pltpu.VMEM((2, C), jnp.int32),
        pltpu.VMEM((2, C, HD), jnp.float32),
        pltpu.VMEM((2, C, HD), jnp.float32),
        pltpu.VMEM((2, C, HD), jnp.float32),
        pltpu.VMEM((2, C, HD), jnp.float32),
        pltpu.VMEM((2, C, 16), jnp.float32),
        pltpu.VMEM_SHARED((NPAD, HD), jnp.float32),
        pltpu.VMEM_SHARED((NPAD, 16), jnp.float32),
        pltpu.SemaphoreType.DMA,
        pltpu.SemaphoreType.DMA,
        pltpu.SemaphoreType.DMA,
        pltpu.SemaphoreType.DMA,
    ],
)(_edge_body)


def _comb_body(aw_ref, az_ref, o_ref):
    wsum = aw_ref[0] + aw_ref[1]
    zsum = az_ref[0] + az_ref[1] + 1e-6
    for hh in range(HEADS):
        o_ref[:, hh * 16:(hh + 1) * 16] = (
            wsum[:, hh * 16:(hh + 1) * 16] / zsum[:, hh:hh + 1])


_COMB_BLK = 2000
_comb_call = pl.pallas_call(
    _comb_body,
    grid=(N // _COMB_BLK,),
    in_specs=[
        pl.BlockSpec((2, _COMB_BLK, HD), lambda i: (0, i, 0)),
        pl.BlockSpec((2, _COMB_BLK, 16), lambda i: (0, i, 0)),
    ],
    out_specs=pl.BlockSpec((_COMB_BLK, HD), lambda i: (i, 0)),
    out_shape=jax.ShapeDtypeStruct((N, HD), jnp.float32),
)


def kernel(h, e, edge_index, Wq, bq, Wk, bk, We, be, Wv, bv):
    q, k, v = _qkv_call(h, Wq, bq.reshape(1, HD), Wk, bk.reshape(1, HD),
                        Wv, bv.reshape(1, HD))
    ee = _ee_call(e, We, be.reshape(1, HD))
    src2 = edge_index[0].reshape(TOTAL_CHUNKS, C)
    dst2 = edge_index[1].reshape(TOTAL_CHUNKS, C)
    accw, accz = _edge_call(q, k, v, ee, src2, dst2)
    out = _comb_call(accw, accz)
    return out.reshape(N, HEADS, D)


# speedup vs baseline: 1.4998x; 1.0009x over previous
import functools

import jax
import jax.numpy as jnp
from jax import lax
from jax.experimental import pallas as pl
from jax.experimental.pallas import tpu as pltpu
from jax.experimental.pallas import tpu_sc as plsc

N = 10000
EDGES = 320000
IN_DIM = 128
HEADS = 8
D = 16
HD = HEADS * D

NUM_WORKERS = 32
NPAD = 10240
C = 32
TOTAL_CHUNKS = EDGES // C
IDX_BLK = 16
NBLOCKS = TOTAL_CHUNKS // IDX_BLK
ROWS_PER_TILE = NPAD // 16


def _qkv_body(h_ref, wq_ref, bq_ref, wk_ref, bk_ref, wv_ref, bv_ref,
              q_ref, k_ref, v_ref):
    hb = h_ref[...]
    f32 = jnp.float32
    q_ref[...] = jnp.dot(hb, wq_ref[...], preferred_element_type=f32) + bq_ref[...]
    k_ref[...] = jnp.dot(hb, wk_ref[...], preferred_element_type=f32) + bk_ref[...]
    v_ref[...] = jnp.dot(hb, wv_ref[...], preferred_element_type=f32) + bv_ref[...]


_QKV_BLK = 2000
_qkv_call = pl.pallas_call(
    _qkv_body,
    grid=(N // _QKV_BLK,),
    in_specs=[
        pl.BlockSpec((_QKV_BLK, IN_DIM), lambda i: (i, 0)),
        pl.BlockSpec((IN_DIM, HD), lambda i: (0, 0)),
        pl.BlockSpec((1, HD), lambda i: (0, 0)),
        pl.BlockSpec((IN_DIM, HD), lambda i: (0, 0)),
        pl.BlockSpec((1, HD), lambda i: (0, 0)),
        pl.BlockSpec((IN_DIM, HD), lambda i: (0, 0)),
        pl.BlockSpec((1, HD), lambda i: (0, 0)),
    ],
    out_specs=[
        pl.BlockSpec((_QKV_BLK, HD), lambda i: (i, 0)),
        pl.BlockSpec((_QKV_BLK, HD), lambda i: (i, 0)),
        pl.BlockSpec((_QKV_BLK, HD), lambda i: (i, 0)),
    ],
    out_shape=[
        jax.ShapeDtypeStruct((N, HD), jnp.float32),
        jax.ShapeDtypeStruct((N, HD), jnp.float32),
        jax.ShapeDtypeStruct((N, HD), jnp.float32),
    ],
)


def _ee_body(e_ref, we_ref, be_ref, o_ref):
    o_ref[...] = (jnp.dot(e_ref[...], we_ref[...],
                          preferred_element_type=jnp.float32) + be_ref[...]) * 0.25


_EE_BLK = 16000
_ee_call = pl.pallas_call(
    _ee_body,
    grid=(EDGES // _EE_BLK,),
    in_specs=[
        pl.BlockSpec((_EE_BLK, IN_DIM), lambda i: (i, 0)),
        pl.BlockSpec((IN_DIM, HD), lambda i: (0, 0)),
        pl.BlockSpec((1, HD), lambda i: (0, 0)),
    ],
    out_specs=pl.BlockSpec((_EE_BLK, HD), lambda i: (i, 0)),
    out_shape=jax.ShapeDtypeStruct((EDGES, HD), jnp.float32),
)


def _edge_body(q_hbm, k_hbm, v_hbm, ee_hbm, src_hbm, dst_hbm, outw, outz,
               src_blk, dst_blk, sdst, krows, qrows, erows, vrows, zrows,
               accw, accz, sem_g0, sem_g1, sem_s0, sem_s1):
    c = lax.axis_index("c")
    s = lax.axis_index("s")
    w = c * 16 + s
    lane = lax.iota(jnp.int32, 16)
    perms = [lane ^ sh for sh in (1, 2, 4, 8)]
    zero16 = jnp.zeros((16,), jnp.float32)
    sem_g = (sem_g0, sem_g1)
    sem_s = (sem_s0, sem_s1)

    def _allsum(v):
        for p in perms:
            v = v + v.at[p].get(mode="promise_in_bounds")
        return v

    zero16i = jnp.zeros((16,), jnp.int32)

    def _zq(i, carry):
        for j in range(HD // 16):
            qrows[0, i, pl.ds(j * 16, 16)] = zero16
            vrows[1, i, pl.ds(j * 16, 16)] = zero16
        zrows[0, i, :] = zero16
        zrows[1, i, :] = zero16
        return carry

    lax.fori_loop(0, C, _zq, 0)
    for j in range(C // 16):
        sdst[1, pl.ds(j * 16, 16)] = zero16i

    row0 = s * ROWS_PER_TILE
    for j in range(ROWS_PER_TILE // C):
        pltpu.sync_copy(qrows.at[0], accw.at[pl.ds(row0 + j * C, C)])
        pltpu.sync_copy(zrows.at[0], accz.at[pl.ds(row0 + j * C, C)])
    plsc.subcore_barrier()

    g0 = (w * NBLOCKS) // NUM_WORKERS
    g1 = ((w + 1) * NBLOCKS) // NUM_WORKERS

    def _reload(t):
        bbase = jnp.minimum(t, TOTAL_CHUNKS - IDX_BLK)
        pltpu.sync_copy(src_hbm.at[pl.ds(bbase, IDX_BLK)], src_blk)
        pltpu.sync_copy(dst_hbm.at[pl.ds(bbase, IDX_BLK)], dst_blk)

    def _issue_gathers(r, t, b):
        pltpu.async_copy(k_hbm.at[src_blk.at[r]], krows.at[b], sem_g[b])
        pltpu.async_copy(q_hbm.at[dst_blk.at[r]], qrows.at[b], sem_g[b])
        pltpu.async_copy(ee_hbm.at[pl.ds(t * C, C)], erows.at[b], sem_g[b])
        pltpu.async_copy(v_hbm.at[src_blk.at[r]], vrows.at[b], sem_g[b])

    def _wait_gathers(r, b):
        pltpu.make_async_copy(k_hbm.at[src_blk.at[r]], krows.at[b], sem_g[b]).wait()
        pltpu.make_async_copy(q_hbm.at[dst_blk.at[r]], qrows.at[b], sem_g[b]).wait()
        pltpu.make_async_copy(ee_hbm.at[pl.ds(0, C)], erows.at[b], sem_g[b]).wait()
        pltpu.make_async_copy(v_hbm.at[src_blk.at[r]], vrows.at[b], sem_g[b]).wait()

    def _snap_dst(r, b):
        for j in range(C // 16):
            sdst[b, pl.ds(j * 16, 16)] = dst_blk[r, pl.ds(j * 16, 16)]

    def _issue_scatter(b):
        pltpu.async_copy(vrows.at[b], accw.at[sdst.at[b]], sem_s[b], add=True)
        pltpu.async_copy(zrows.at[b], accz.at[sdst.at[b]], sem_s[b], add=True)

    def _dummy_scatter():
        pltpu.async_copy(vrows.at[1], accw.at[sdst.at[1]], sem_s[1], add=True)
        pltpu.async_copy(zrows.at[1], accz.at[sdst.at[1]], sem_s[1], add=True)

    def _wait_scatter(b):
        pltpu.make_async_copy(vrows.at[b], accw.at[sdst.at[b]], sem_s[b]).wait()
        pltpu.make_async_copy(zrows.at[b], accz.at[sdst.at[b]], sem_s[b]).wait()

    def _compute(b):
        kr, qr, er, vr, zr = (krows.at[b], qrows.at[b], erows.at[b],
                              vrows.at[b], zrows.at[b])

        def _edge(i):
            zrow = zero16
            for hh in range(HEADS):
                sl = pl.ds(hh * 16, 16)
                t3 = kr[i, sl] * qr[i, sl] * er[i, sl]
                sv = jnp.exp(jnp.minimum(jnp.maximum(_allsum(t3), -5.0), 5.0))
                vr[i, sl] = vr[i, sl] * sv
                zrow = jnp.where(lane == hh, sv, zrow)
            zr[i, :] = zrow

        plsc.parallel_loop(0, C, 1, unroll=1)(_edge)

    _dummy_scatter()
    _reload(g0 * IDX_BLK)
    _issue_gathers(0, g0 * IDX_BLK, 0)

    def _block(g, carry):
        tbase = g * IDX_BLK
        for r in range(IDX_BLK):
            b = r & 1
            nb = 1 - b
            _wait_gathers(r, b)
            _snap_dst(r, b)
            _wait_scatter(nb)
            if r == IDX_BLK - 1:
                _reload(tbase + IDX_BLK)
            tn = jnp.minimum(tbase + r + 1, TOTAL_CHUNKS - 1)
            _issue_gathers((r + 1) % IDX_BLK, tn, nb)
            _compute(b)
            _issue_scatter(b)
        return carry

    lax.fori_loop(g0, g1, _block, 0)

    _wait_gathers(0, 0)
    _wait_scatter(1)

    plsc.subcore_barrier()

    for j in range(ROWS_PER_TILE // C):
        pltpu.sync_copy(accw.at[pl.ds(row0 + j * C, C)], qrows.at[0])
        pltpu.sync_copy(qrows.at[0], outw.at[c, pl.ds(row0 + j * C, C)])
        pltpu.sync_copy(accz.at[pl.ds(row0 + j * C, C)], zrows.at[0])
        pltpu.sync_copy(zrows.at[0], outz.at[c, pl.ds(row0 + j * C, C)])


_edge_call = functools.partial(
    pl.kernel,
    out_type=(
        jax.ShapeDtypeStruct((2, NPAD, HD), jnp.float32),
        jax.ShapeDtypeStruct((2, NPAD, 16), jnp.float32),
    ),
    mesh=plsc.VectorSubcoreMesh(core_axis_name="c", subcore_axis_name="s"),
    compiler_params=pltpu.CompilerParams(use_tc_tiling_on_sc=False),
    scratch_types=[
        pltpu.VMEM((IDX_BLK, C), jnp.int32),
        pltpu.VMEM((IDX_BLK, C), jnp.int32),
        pltpu.VMEM((2, C), jnp.int32),
        pltpu.VMEM((2, C, HD), jnp.float32),
        pltpu.VMEM((2, C, HD), jnp.float32),
        pltpu.VMEM((2, C, HD), jnp.float32),
        pltpu.VMEM((2, C, HD), jnp.float32),
        pltpu.VMEM((2, C, 16), jnp.float32),
        pltpu.VMEM_SHARED((NPAD, HD), jnp.float32),
        pltpu.VMEM_SHARED((NPAD, 16), jnp.float32),
        pltpu.SemaphoreType.DMA,
        pltpu.SemaphoreType.DMA,
        pltpu.SemaphoreType.DMA,
        pltpu.SemaphoreType.DMA,
    ],
)(_edge_body)


def _comb_body(aw_ref, az_ref, o_ref):
    wsum = aw_ref[0] + aw_ref[1]
    zsum = az_ref[0] + az_ref[1] + 1e-6
    for hh in range(HEADS):
        o_ref[:, hh * 16:(hh + 1) * 16] = (
            wsum[:, hh * 16:(hh + 1) * 16] / zsum[:, hh:hh + 1])


_COMB_BLK = 2000
_comb_call = pl.pallas_call(
    _comb_body,
    grid=(N // _COMB_BLK,),
    in_specs=[
        pl.BlockSpec((2, _COMB_BLK, HD), lambda i: (0, i, 0)),
        pl.BlockSpec((2, _COMB_BLK, 16), lambda i: (0, i, 0)),
    ],
    out_specs=pl.BlockSpec((_COMB_BLK, HD), lambda i: (i, 0)),
    out_shape=jax.ShapeDtypeStruct((N, HD), jnp.float32),
)


def kernel(h, e, edge_index, Wq, bq, Wk, bk, We, be, Wv, bv):
    q, k, v = _qkv_call(h, Wq, bq.reshape(1, HD), Wk, bk.reshape(1, HD),
                        Wv, bv.reshape(1, HD))
    ee = _ee_call(e, We, be.reshape(1, HD))
    src2 = edge_index[0].reshape(TOTAL_CHUNKS, C)
    dst2 = edge_index[1].reshape(TOTAL_CHUNKS, C)
    accw, accz = _edge_call(q, k, v, ee, src2, dst2)
    out = _comb_call(accw, accz)
    return out.reshape(N, HEADS, D)
